# baseline (speedup 1.0000x reference)
import functools

import jax
import jax.numpy as jnp
from jax import lax
from jax.experimental import pallas as pl
from jax.experimental.pallas import tpu as pltpu
from jax.experimental.pallas import tpu_sc as plsc

_N_CELLS = 50000
_N_GENES = 128
_N_LABELS = 8
_EMB = 64
_B = 16384

_BKC = 4096


def _tc_tables_body(cells_ref, genes_ref, embt_ref, ctw_ref, decw_ref,
                    decb_ref, p_ref, qidx_ref, pcr_ref):
    i = pl.program_id(0)
    ctw = ctw_ref[...]

    @pl.when(i == 0)
    def _():
        for s in range(8):
            logits = lax.dot_general(
                embt_ref[:, s * 6250:(s + 1) * 6250], ctw,
                (((0,), (1,)), ((), ())),
                preferred_element_type=jnp.float32)
            m = jnp.max(logits, axis=1, keepdims=True)
            e = jnp.exp(logits - m)
            p = e / jnp.sum(e, axis=1, keepdims=True)
            p_ref[0:6250, s * 16:(s + 1) * 16] = jnp.concatenate([p, p], 1)

    @pl.when(i < _B // _BKC)
    def _():
        c = cells_ref[0, pl.ds(i * _BKC, _BKC)]
        g = genes_ref[0, pl.ds(i * _BKC, _BKC)]
        l8 = lax.broadcasted_iota(jnp.int32, (_BKC, _N_LABELS), 1)
        qidx_ref[...] = c[:, None] * 1024 + l8 * 128 + g[:, None]

        dec = lax.dot_general(
            ctw, decw_ref[...], (((1,), (1,)), ((), ())),
            preferred_element_type=jnp.float32)
        dec = dec + decb_ref[...]
        m0 = jnp.max(dec, axis=1, keepdims=True)
        e0 = jnp.exp(dec - m0)
        pcT = e0 / jnp.sum(e0, axis=1, keepdims=True)

        oh = (g[:, None]
              == lax.broadcasted_iota(jnp.int32, (_BKC, _N_GENES), 1))
        pcr = lax.dot_general(
            oh.astype(jnp.float32), pcT, (((1,), (1,)), ((), ())),
            preferred_element_type=jnp.float32)
        pcr_ref[...] = jnp.concatenate([pcr, pcr], axis=1)


def _tc_tables(cells, genes, emb_t, ct_W, dec_W, dec_b):
    grid = _B // _BKC
    return pl.pallas_call(
        _tc_tables_body,
        grid=(grid,),
        in_specs=[
            pl.BlockSpec((1, _B), lambda i: (0, 0)),
            pl.BlockSpec((1, _B), lambda i: (0, 0)),
            pl.BlockSpec((_EMB, _N_CELLS), lambda i: (0, 0)),
            pl.BlockSpec((_N_LABELS, _EMB), lambda i: (0, 0)),
            pl.BlockSpec((_N_GENES, _EMB), lambda i: (0, 0)),
            pl.BlockSpec((1, _N_GENES), lambda i: (0, 0)),
        ],
        out_specs=[
            pl.BlockSpec((6256, 128), lambda i: (0, 0)),
            pl.BlockSpec((_BKC, _N_LABELS), lambda i: (i, 0)),
            pl.BlockSpec((_BKC, 2 * _N_LABELS), lambda i: (i, 0)),
        ],
        out_shape=[
            jax.ShapeDtypeStruct((6256, 128), jnp.float32),
            jax.ShapeDtypeStruct((_B, _N_LABELS), jnp.int32),
            jax.ShapeDtypeStruct((_B, 2 * _N_LABELS), jnp.float32),
        ],
    )(cells.reshape(1, _B), genes.reshape(1, _B), emb_t, ct_W, dec_W,
      dec_b.reshape(1, _N_GENES))


def _make_sc_gather():
    info = plsc.get_sparse_core_info()
    nc, ns, nl = info.num_cores, info.num_subcores, info.num_lanes
    nw = nc * ns
    bw = _B // nw
    nlb = _N_LABELS
    rows = bw * nlb // nl

    mesh = plsc.VectorSubcoreMesh(core_axis_name="c", subcore_axis_name="s")

    scratch = [
        pltpu.VMEM((bw,), jnp.int32),
        pltpu.VMEM((bw,), jnp.int32),
        pltpu.VMEM((bw * nlb,), jnp.int32),
        pltpu.VMEM((bw, nl), jnp.float32),
        pltpu.VMEM((bw, nl), jnp.float32),
        pltpu.VMEM((bw * nlb,), jnp.float32),
        pltpu.VMEM((bw, nl), jnp.float32),
        pltpu.SemaphoreType.DMA,
    ]

    @functools.partial(
        pl.kernel,
        mesh=mesh,
        compiler_params=pltpu.CompilerParams(use_tc_tiling_on_sc=False),
        out_type=[
            jax.ShapeDtypeStruct((_B, nl), jnp.float32),
            jax.ShapeDtypeStruct((_B * nlb,), jnp.float32),
            jax.ShapeDtypeStruct((_B, nl), jnp.float32),
        ],
        scratch_types=scratch,
    )
    def sc_kernel(cells_hbm, qidx_hbm, pcr_hbm, pzcw_hbm, p_hbm,
                  pcz_out, qz_out, pz_out,
                  cells_v, krow_v, qidx_v, pz_v, pcr_v, qz_v, pcz_v, sem):
        wid = lax.axis_index("s") * nc + lax.axis_index("c")
        base = wid * bw

        pltpu.sync_copy(cells_hbm.at[pl.ds(base, bw)], cells_v)

        def kr_body(k, _):
            sl = pl.ds(k * nl, nl)
            c = cells_v[sl]
            krow_v[sl] = lax.rem(c, 6250) * 8 + lax.div(c, 6250)
            return 0
        lax.fori_loop(0, bw // nl, kr_body, 0)

        cp_pz = pltpu.async_copy(p_hbm.at[krow_v], pz_v, sem)

        pltpu.sync_copy(qidx_hbm.at[pl.ds(base * nlb, bw * nlb)], qidx_v)
        cp_qz = pltpu.async_copy(pzcw_hbm.at[qidx_v], qz_v, sem)
        cp_pcr = pltpu.async_copy(pcr_hbm.at[pl.ds(base, bw)], pcr_v, sem)

        cp_pz.wait()
        cp_pcr.wait()

        def mul_body(k, _):
            pcz_v[k, :] = pz_v[k, :] * pcr_v[k, :]
            return 0
        lax.fori_loop(0, bw, mul_body, 0)

        pltpu.sync_copy(pz_v, pz_out.at[pl.ds(base, bw)])
        pltpu.sync_copy(pcz_v, pcz_out.at[pl.ds(base, bw)])
        cp_qz.wait()
        pltpu.sync_copy(qz_v, qz_out.at[pl.ds(base * nlb, bw * nlb)])

    return sc_kernel


def kernel(cells, genes, emb_table, ct_W, dec_W, dec_b, pz_cw):
    cells = cells.astype(jnp.int32)
    genes = genes.astype(jnp.int32)
    emb_t = emb_table.T
    pzcw_flat = jnp.transpose(pz_cw, (0, 2, 1)).reshape(-1)
    p2, qidx, pc_rows = _tc_tables(cells, genes, emb_t, ct_W, dec_W, dec_b)
    p_table = p2.reshape(6256 * 8, 16)
    sc = _make_sc_gather()
    pcz16, qz_f, pz16 = sc(cells, qidx.reshape(-1), pc_rows, pzcw_flat,
                           p_table)
    shp = (_B, _N_LABELS)
    return (pcz16[:, :_N_LABELS], qz_f.reshape(shp), pz16[:, :_N_LABELS])


# speedup vs baseline: 1.2060x; 1.2060x over previous
import functools

import jax
import jax.numpy as jnp
from jax import lax
from jax.experimental import pallas as pl
from jax.experimental.pallas import tpu as pltpu
from jax.experimental.pallas import tpu_sc as plsc

_N_CELLS = 50000
_N_GENES = 128
_N_LABELS = 8
_EMB = 64
_B = 16384

_BKC = 4096


def _tc_tables_body(cells_ref, genes_ref, embt_ref, ctw_ref, decw_ref,
                    decb_ref, p_ref, qidx_ref, pcr_ref):
    i = pl.program_id(0)
    ctw = ctw_ref[...]

    @pl.when(i == 0)
    def _():
        for s in range(8):
            w = 6272 if s < 7 else _N_CELLS - 7 * 6272
            logits = lax.dot_general(
                embt_ref[:, s * 6272:s * 6272 + w], ctw,
                (((0,), (1,)), ((), ())),
                preferred_element_type=jnp.float32)
            m = jnp.max(logits, axis=1, keepdims=True)
            e = jnp.exp(logits - m)
            p = e / jnp.sum(e, axis=1, keepdims=True)
            p_ref[0:w, s * 16:(s + 1) * 16] = jnp.concatenate([p, p], 1)

    @pl.when(i < _B // _BKC)
    def _():
        c = cells_ref[0, pl.ds(i * _BKC, _BKC)]
        g = genes_ref[0, pl.ds(i * _BKC, _BKC)]
        l8 = lax.broadcasted_iota(jnp.int32, (_BKC, _N_LABELS), 1)
        qidx_ref[...] = c[:, None] * 1024 + l8 * 128 + g[:, None]

        dec = lax.dot_general(
            ctw, decw_ref[...], (((1,), (1,)), ((), ())),
            preferred_element_type=jnp.float32)
        dec = dec + decb_ref[...]
        m0 = jnp.max(dec, axis=1, keepdims=True)
        e0 = jnp.exp(dec - m0)
        pcT = e0 / jnp.sum(e0, axis=1, keepdims=True)

        oh = (g[:, None]
              == lax.broadcasted_iota(jnp.int32, (_BKC, _N_GENES), 1))
        pcr = lax.dot_general(
            oh.astype(jnp.float32), pcT, (((1,), (1,)), ((), ())),
            preferred_element_type=jnp.float32)
        pcr_ref[...] = jnp.concatenate([pcr, pcr], axis=1)


def _tc_tables(cells, genes, emb_t, ct_W, dec_W, dec_b):
    grid = _B // _BKC
    return pl.pallas_call(
        _tc_tables_body,
        grid=(grid,),
        in_specs=[
            pl.BlockSpec((1, _B), lambda i: (0, 0)),
            pl.BlockSpec((1, _B), lambda i: (0, 0)),
            pl.BlockSpec((_EMB, _N_CELLS), lambda i: (0, 0)),
            pl.BlockSpec((_N_LABELS, _EMB), lambda i: (0, 0)),
            pl.BlockSpec((_N_GENES, _EMB), lambda i: (0, 0)),
            pl.BlockSpec((1, _N_GENES), lambda i: (0, 0)),
        ],
        out_specs=[
            pl.BlockSpec((6272, 128), lambda i: (0, 0)),
            pl.BlockSpec((_BKC, _N_LABELS), lambda i: (i, 0)),
            pl.BlockSpec((_BKC, 2 * _N_LABELS), lambda i: (i, 0)),
        ],
        out_shape=[
            jax.ShapeDtypeStruct((6272, 128), jnp.float32),
            jax.ShapeDtypeStruct((_B, _N_LABELS), jnp.int32),
            jax.ShapeDtypeStruct((_B, 2 * _N_LABELS), jnp.float32),
        ],
    )(cells.reshape(1, _B), genes.reshape(1, _B), emb_t, ct_W, dec_W,
      dec_b.reshape(1, _N_GENES))


def _make_sc_gather():
    info = plsc.get_sparse_core_info()
    nc, ns, nl = info.num_cores, info.num_subcores, info.num_lanes
    nw = nc * ns
    bw = _B // nw
    nlb = _N_LABELS
    rows = bw * nlb // nl

    mesh = plsc.VectorSubcoreMesh(core_axis_name="c", subcore_axis_name="s")

    scratch = [
        pltpu.VMEM((bw,), jnp.int32),
        pltpu.VMEM((bw,), jnp.int32),
        pltpu.VMEM((bw * nlb,), jnp.int32),
        pltpu.VMEM((bw, nl), jnp.float32),
        pltpu.VMEM((bw, nl), jnp.float32),
        pltpu.VMEM((bw * nlb,), jnp.float32),
        pltpu.VMEM((bw, nl), jnp.float32),
        pltpu.SemaphoreType.DMA,
    ]

    @functools.partial(
        pl.kernel,
        mesh=mesh,
        compiler_params=pltpu.CompilerParams(use_tc_tiling_on_sc=False),
        out_type=[
            jax.ShapeDtypeStruct((_B, nl), jnp.float32),
            jax.ShapeDtypeStruct((_B * nlb,), jnp.float32),
            jax.ShapeDtypeStruct((_B, nl), jnp.float32),
        ],
        scratch_types=scratch,
    )
    def sc_kernel(cells_hbm, qidx_hbm, pcr_hbm, pzcw_hbm, p_hbm,
                  pcz_out, qz_out, pz_out,
                  cells_v, krow_v, qidx_v, pz_v, pcr_v, qz_v, pcz_v, sem):
        wid = lax.axis_index("s") * nc + lax.axis_index("c")
        base = wid * bw

        pltpu.sync_copy(cells_hbm.at[pl.ds(base, bw)], cells_v)

        def kr_body(k, _):
            sl = pl.ds(k * nl, nl)
            c = cells_v[sl]
            krow_v[sl] = lax.rem(c, 6272) * 8 + lax.div(c, 6272)
            return 0
        lax.fori_loop(0, bw // nl, kr_body, 0)

        cp_pz = pltpu.async_copy(p_hbm.at[krow_v], pz_v, sem)

        pltpu.sync_copy(qidx_hbm.at[pl.ds(base * nlb, bw * nlb)], qidx_v)
        cp_qz = pltpu.async_copy(pzcw_hbm.at[qidx_v], qz_v, sem)
        cp_pcr = pltpu.async_copy(pcr_hbm.at[pl.ds(base, bw)], pcr_v, sem)

        cp_pz.wait()
        cp_pcr.wait()

        def mul_body(k, _):
            pcz_v[k, :] = pz_v[k, :] * pcr_v[k, :]
            return 0
        lax.fori_loop(0, bw, mul_body, 0)

        pltpu.sync_copy(pz_v, pz_out.at[pl.ds(base, bw)])
        pltpu.sync_copy(pcz_v, pcz_out.at[pl.ds(base, bw)])
        cp_qz.wait()
        pltpu.sync_copy(qz_v, qz_out.at[pl.ds(base * nlb, bw * nlb)])

    return sc_kernel


def kernel(cells, genes, emb_table, ct_W, dec_W, dec_b, pz_cw):
    cells = cells.astype(jnp.int32)
    genes = genes.astype(jnp.int32)
    emb_t = emb_table.T
    pzcw_flat = jnp.transpose(pz_cw, (0, 2, 1)).reshape(-1)
    p2, qidx, pc_rows = _tc_tables(cells, genes, emb_t, ct_W, dec_W, dec_b)
    p_table = p2.reshape(6272 * 8, 16)
    sc = _make_sc_gather()
    pcz16, qz_f, pz16 = sc(cells, qidx.reshape(-1), pc_rows, pzcw_flat,
                           p_table)
    shp = (_B, _N_LABELS)
    return (pcz16[:, :_N_LABELS], qz_f.reshape(shp), pz16[:, :_N_LABELS])


# speedup vs baseline: 1.3819x; 1.1458x over previous
import functools

import jax
import jax.numpy as jnp
from jax import lax
from jax.experimental import pallas as pl
from jax.experimental.pallas import tpu as pltpu
from jax.experimental.pallas import tpu_sc as plsc

_N_CELLS = 50000
_N_GENES = 128
_N_LABELS = 8
_EMB = 64
_B = 16384

_PS = 6272


def _tc_tables_body(embt_ref, ctw_ref, decw_ref, decb_ref, p_ref, pc_ref):
    ctw = ctw_ref[...]
    for s in range(8):
        w = _PS if s < 7 else _N_CELLS - 7 * _PS
        logits = lax.dot_general(
            embt_ref[:, s * _PS:s * _PS + w], ctw,
            (((0,), (1,)), ((), ())),
            preferred_element_type=jnp.float32)
        m = jnp.max(logits, axis=1, keepdims=True)
        e = jnp.exp(logits - m)
        p = e / jnp.sum(e, axis=1, keepdims=True)
        p_ref[0:w, s * 16:(s + 1) * 16] = jnp.concatenate([p, p], 1)

    dec = lax.dot_general(
        decw_ref[...], ctw, (((1,), (1,)), ((), ())),
        preferred_element_type=jnp.float32)
    dec = dec + decb_ref[...].reshape(_N_GENES, 1)
    m0 = jnp.max(dec, axis=0, keepdims=True)
    e0 = jnp.exp(dec - m0)
    pc = e0 / jnp.sum(e0, axis=0, keepdims=True)
    pc_ref[...] = jnp.concatenate([pc, pc], axis=1)


def _tc_tables(emb_t, ct_W, dec_W, dec_b):
    return pl.pallas_call(
        _tc_tables_body,
        grid=(1,),
        in_specs=[
            pl.BlockSpec((_EMB, _N_CELLS), lambda i: (0, 0)),
            pl.BlockSpec((_N_LABELS, _EMB), lambda i: (0, 0)),
            pl.BlockSpec((_N_GENES, _EMB), lambda i: (0, 0)),
            pl.BlockSpec((1, _N_GENES), lambda i: (0, 0)),
        ],
        out_specs=[
            pl.BlockSpec((_PS, 128), lambda i: (0, 0)),
            pl.BlockSpec((_N_GENES, 16), lambda i: (0, 0)),
        ],
        out_shape=[
            jax.ShapeDtypeStruct((_PS, 128), jnp.float32),
            jax.ShapeDtypeStruct((_N_GENES, 16), jnp.float32),
        ],
    )(emb_t, ct_W, dec_W, dec_b.reshape(1, _N_GENES))


def _take16(win, idx):
    return jnp.take_along_axis(win, idx, axis=0, mode="promise_in_bounds")


def _make_sc_gather():
    info = plsc.get_sparse_core_info()
    nc, ns, nl = info.num_cores, info.num_subcores, info.num_lanes
    nw = nc * ns
    bw = _B // nw
    nlb = _N_LABELS

    mesh = plsc.VectorSubcoreMesh(core_axis_name="c", subcore_axis_name="s")

    scratch = [
        pltpu.VMEM((bw,), jnp.int32),
        pltpu.VMEM((bw,), jnp.int32),
        pltpu.VMEM((bw,), jnp.int32),
        pltpu.VMEM((bw * nlb,), jnp.int32),
        pltpu.VMEM((bw, nl), jnp.float32),
        pltpu.VMEM((bw, nl), jnp.float32),
        pltpu.VMEM((bw * nlb,), jnp.float32),
        pltpu.VMEM((bw, nl), jnp.float32),
        pltpu.SemaphoreType.DMA,
        pltpu.SemaphoreType.DMA,
    ]

    @functools.partial(
        pl.kernel,
        mesh=mesh,
        compiler_params=pltpu.CompilerParams(use_tc_tiling_on_sc=False),
        out_type=[
            jax.ShapeDtypeStruct((_B, nl), jnp.float32),
            jax.ShapeDtypeStruct((_B * nlb,), jnp.float32),
            jax.ShapeDtypeStruct((_B, nl), jnp.float32),
        ],
        scratch_types=scratch,
    )
    def sc_kernel(cells_hbm, genes_hbm, pzcw_hbm, p_hbm, pc_hbm,
                  pcz_out, qz_out, pz_out,
                  cells_v, genes_v, krow_v, qidx_v, pz_v, pcr_v, qz_v,
                  pcz_v, sem, sem2):
        wid = lax.axis_index("s") * nc + lax.axis_index("c")
        base = wid * bw

        pltpu.sync_copy(cells_hbm.at[pl.ds(base, bw)], cells_v)
        pltpu.sync_copy(genes_hbm.at[pl.ds(base, bw)], genes_v)
        cp_pcr = pltpu.async_copy(pc_hbm.at[genes_v], pcr_v, sem2)

        def kr_body(k, _):
            sl = pl.ds(k * nl, nl)
            c = cells_v[sl]
            krow_v[sl] = lax.rem(c, _PS) * 8 + lax.div(c, _PS)
            return 0
        lax.fori_loop(0, bw // nl, kr_body, 0)

        cp_pz = pltpu.async_copy(p_hbm.at[krow_v], pz_v, sem)

        i16 = lax.iota(jnp.int32, nl)
        lsub = i16 & (nlb - 1)
        esub = i16 >> 3

        def qidx_body(k, _):
            win = pl.ds((k >> 3) * nl, nl)
            erl = esub + (k & 7) * 2
            c16 = _take16(cells_v[win], erl)
            g16 = _take16(genes_v[win], erl)
            qidx_v[pl.ds(k * nl, nl)] = c16 * 1024 + lsub * 128 + g16
            return 0
        lax.fori_loop(0, (bw * nlb) // nl, qidx_body, 0)

        cp_qz = pltpu.async_copy(pzcw_hbm.at[qidx_v], qz_v, sem)

        cp_pz.wait()
        cp_pcr.wait()

        def mul_body(k, _):
            pcz_v[k, :] = pz_v[k, :] * pcr_v[k, :]
            return 0
        lax.fori_loop(0, bw, mul_body, 0)

        pltpu.sync_copy(pz_v, pz_out.at[pl.ds(base, bw)])
        pltpu.sync_copy(pcz_v, pcz_out.at[pl.ds(base, bw)])
        cp_qz.wait()
        pltpu.sync_copy(qz_v, qz_out.at[pl.ds(base * nlb, bw * nlb)])

    return sc_kernel


def kernel(cells, genes, emb_table, ct_W, dec_W, dec_b, pz_cw):
    cells = cells.astype(jnp.int32)
    genes = genes.astype(jnp.int32)
    emb_t = emb_table.T
    pzcw_flat = jnp.transpose(pz_cw, (0, 2, 1)).reshape(-1)
    p2, pc16 = _tc_tables(emb_t, ct_W, dec_W, dec_b)
    p_table = p2.reshape(_PS * 8, 16)
    sc = _make_sc_gather()
    pcz16, qz_f, pz16 = sc(cells, genes, pzcw_flat, p_table, pc16)
    shp = (_B, _N_LABELS)
    return (pcz16[:, :_N_LABELS], qz_f.reshape(shp), pz16[:, :_N_LABELS])


# speedup vs baseline: 1.7566x; 1.2712x over previous
import functools

import jax
import jax.numpy as jnp
from jax import lax
from jax.experimental import pallas as pl
from jax.experimental.pallas import tpu as pltpu
from jax.experimental.pallas import tpu_sc as plsc

_N_CELLS = 50000
_N_GENES = 128
_N_LABELS = 8
_EMB = 64
_B = 16384

_PS = 6272


def _tc_tables_body(embt_ref, ctw_ref, decw_ref, decb_ref, p_ref, pc_ref):
    ctw = ctw_ref[...]
    dup = (lax.broadcasted_iota(jnp.int32, (_N_LABELS, 16), 1) & 7
           == lax.broadcasted_iota(jnp.int32, (_N_LABELS, 16), 0)
           ).astype(jnp.float32)
    for s in range(8):
        w = _PS if s < 7 else _N_CELLS - 7 * _PS
        logits = lax.dot_general(
            ctw, embt_ref[:, s * _PS:s * _PS + w],
            (((1,), (0,)), ((), ())),
            preferred_element_type=jnp.float32)
        m = jnp.max(logits, axis=0, keepdims=True)
        e = jnp.exp(logits - m)
        pT = e / jnp.sum(e, axis=0, keepdims=True)
        p_ref[0:w, s * 16:(s + 1) * 16] = lax.dot_general(
            pT, dup, (((0,), (0,)), ((), ())),
            preferred_element_type=jnp.float32)

    dec = lax.dot_general(
        decw_ref[...], ctw, (((1,), (1,)), ((), ())),
        preferred_element_type=jnp.float32)
    dec = dec + decb_ref[...].reshape(_N_GENES, 1)
    m0 = jnp.max(dec, axis=0, keepdims=True)
    e0 = jnp.exp(dec - m0)
    pc = e0 / jnp.sum(e0, axis=0, keepdims=True)
    pc_ref[...] = jnp.concatenate([pc, pc], axis=1)


def _tc_tables(emb_t, ct_W, dec_W, dec_b):
    return pl.pallas_call(
        _tc_tables_body,
        grid=(1,),
        in_specs=[
            pl.BlockSpec((_EMB, _N_CELLS), lambda i: (0, 0)),
            pl.BlockSpec((_N_LABELS, _EMB), lambda i: (0, 0)),
            pl.BlockSpec((_N_GENES, _EMB), lambda i: (0, 0)),
            pl.BlockSpec((1, _N_GENES), lambda i: (0, 0)),
        ],
        out_specs=[
            pl.BlockSpec((_PS, 128), lambda i: (0, 0)),
            pl.BlockSpec((_N_GENES, 16), lambda i: (0, 0)),
        ],
        out_shape=[
            jax.ShapeDtypeStruct((_PS, 128), jnp.float32),
            jax.ShapeDtypeStruct((_N_GENES, 16), jnp.float32),
        ],
    )(emb_t, ct_W, dec_W, dec_b.reshape(1, _N_GENES))


def _take16(win, idx):
    return jnp.take_along_axis(win, idx, axis=0, mode="promise_in_bounds")


def _make_sc_gather():
    info = plsc.get_sparse_core_info()
    nc, ns, nl = info.num_cores, info.num_subcores, info.num_lanes
    nw = nc * ns
    bw = _B // nw
    nlb = _N_LABELS

    mesh = plsc.VectorSubcoreMesh(core_axis_name="c", subcore_axis_name="s")

    scratch = [
        pltpu.VMEM((bw,), jnp.int32),
        pltpu.VMEM((bw,), jnp.int32),
        pltpu.VMEM((bw,), jnp.int32),
        pltpu.VMEM((bw * nlb,), jnp.int32),
        pltpu.VMEM((bw, nl), jnp.float32),
        pltpu.VMEM((bw, nl), jnp.float32),
        pltpu.VMEM((bw * nlb,), jnp.float32),
        pltpu.VMEM((bw, nl), jnp.float32),
        pltpu.SemaphoreType.DMA,
        pltpu.SemaphoreType.DMA,
    ]

    @functools.partial(
        pl.kernel,
        mesh=mesh,
        compiler_params=pltpu.CompilerParams(use_tc_tiling_on_sc=False),
        out_type=[
            jax.ShapeDtypeStruct((_B, nl), jnp.float32),
            jax.ShapeDtypeStruct((_B * nlb,), jnp.float32),
            jax.ShapeDtypeStruct((_B, nl), jnp.float32),
        ],
        scratch_types=scratch,
    )
    def sc_kernel(cells_hbm, genes_hbm, pzcw_hbm, p_hbm, pc_hbm,
                  pcz_out, qz_out, pz_out,
                  cells_v, genes_v, krow_v, qidx_v, pz_v, pcr_v, qz_v,
                  pcz_v, sem, sem2):
        wid = lax.axis_index("s") * nc + lax.axis_index("c")
        base = wid * bw

        pltpu.sync_copy(cells_hbm.at[pl.ds(base, bw)], cells_v)
        pltpu.sync_copy(genes_hbm.at[pl.ds(base, bw)], genes_v)
        cp_pcr = pltpu.async_copy(pc_hbm.at[genes_v], pcr_v, sem2)

        def kr_body(k, _):
            sl = pl.ds(k * nl, nl)
            c = cells_v[sl]
            krow_v[sl] = lax.rem(c, _PS) * 8 + lax.div(c, _PS)
            return 0
        lax.fori_loop(0, bw // nl, kr_body, 0)

        cp_pz = pltpu.async_copy(p_hbm.at[krow_v], pz_v, sem)

        i16 = lax.iota(jnp.int32, nl)
        lsub = i16 & (nlb - 1)
        esub = i16 >> 3

        def qidx_body(k, _):
            win = pl.ds((k >> 3) * nl, nl)
            erl = esub + (k & 7) * 2
            c16 = _take16(cells_v[win], erl)
            g16 = _take16(genes_v[win], erl)
            qidx_v[pl.ds(k * nl, nl)] = c16 * 1024 + lsub * 128 + g16
            return 0
        lax.fori_loop(0, (bw * nlb) // nl, qidx_body, 0)

        cp_qz = pltpu.async_copy(pzcw_hbm.at[qidx_v], qz_v, sem)

        cp_pz.wait()
        cp_pcr.wait()

        def mul_body(k, _):
            pcz_v[k, :] = pz_v[k, :] * pcr_v[k, :]
            return 0
        lax.fori_loop(0, bw, mul_body, 0)

        pltpu.sync_copy(pz_v, pz_out.at[pl.ds(base, bw)])
        pltpu.sync_copy(pcz_v, pcz_out.at[pl.ds(base, bw)])
        cp_qz.wait()
        pltpu.sync_copy(qz_v, qz_out.at[pl.ds(base * nlb, bw * nlb)])

    return sc_kernel


def kernel(cells, genes, emb_table, ct_W, dec_W, dec_b, pz_cw):
    cells = cells.astype(jnp.int32)
    genes = genes.astype(jnp.int32)
    emb_t = emb_table.T
    pzcw_flat = jnp.transpose(pz_cw, (0, 2, 1)).reshape(-1)
    p2, pc16 = _tc_tables(emb_t, ct_W, dec_W, dec_b)
    p_table = p2.reshape(_PS * 8, 16)
    sc = _make_sc_gather()
    pcz16, qz_f, pz16 = sc(cells, genes, pzcw_flat, p_table, pc16)
    shp = (_B, _N_LABELS)
    return (pcz16[:, :_N_LABELS], qz_f.reshape(shp), pz16[:, :_N_LABELS])


# speedup vs baseline: 1.8871x; 1.0742x over previous
import functools

import jax
import jax.numpy as jnp
from jax import lax
from jax.experimental import pallas as pl
from jax.experimental.pallas import tpu as pltpu
from jax.experimental.pallas import tpu_sc as plsc

_N_CELLS = 50000
_N_GENES = 128
_N_LABELS = 8
_EMB = 64
_B = 16384

_PS = 6272


def _tc_tables_body(embt_ref, ctw_ref, decw_ref, decb_ref, p_ref, pc_ref):
    ctw = ctw_ref[...]
    dup = (lax.broadcasted_iota(jnp.int32, (_N_LABELS, 16), 1) & 7
           == lax.broadcasted_iota(jnp.int32, (_N_LABELS, 16), 0)
           ).astype(jnp.float32)
    for s in range(8):
        w = _PS if s < 7 else _N_CELLS - 7 * _PS
        logits = lax.dot_general(
            ctw, embt_ref[:, s * _PS:s * _PS + w],
            (((1,), (0,)), ((), ())),
            preferred_element_type=jnp.float32)
        m = jnp.max(logits, axis=0, keepdims=True)
        e = jnp.exp(logits - m)
        pT = e / jnp.sum(e, axis=0, keepdims=True)
        p_ref[0:w, s * 16:(s + 1) * 16] = lax.dot_general(
            pT, dup, (((0,), (0,)), ((), ())),
            preferred_element_type=jnp.float32)

    decT = lax.dot_general(
        ctw, decw_ref[...], (((1,), (1,)), ((), ())),
        preferred_element_type=jnp.float32)
    decT = decT + decb_ref[...]
    m0 = jnp.max(decT, axis=1, keepdims=True)
    e0 = jnp.exp(decT - m0)
    pcT = e0 / jnp.sum(e0, axis=1, keepdims=True)
    dup128 = (lax.broadcasted_iota(jnp.int32, (_N_LABELS, 128), 1) & 7
              == lax.broadcasted_iota(jnp.int32, (_N_LABELS, 128), 0)
              ).astype(jnp.float32)
    pc_ref[...] = lax.dot_general(
        pcT, dup128, (((0,), (0,)), ((), ())),
        preferred_element_type=jnp.float32)


def _tc_tables(emb_t, ct_W, dec_W, dec_b):
    return pl.pallas_call(
        _tc_tables_body,
        grid=(1,),
        in_specs=[
            pl.BlockSpec((_EMB, _N_CELLS), lambda i: (0, 0)),
            pl.BlockSpec((_N_LABELS, _EMB), lambda i: (0, 0)),
            pl.BlockSpec((_N_GENES, _EMB), lambda i: (0, 0)),
            pl.BlockSpec((1, _N_GENES), lambda i: (0, 0)),
        ],
        out_specs=[
            pl.BlockSpec((_PS, 128), lambda i: (0, 0)),
            pl.BlockSpec((_N_GENES, 128), lambda i: (0, 0)),
        ],
        out_shape=[
            jax.ShapeDtypeStruct((_PS, 128), jnp.float32),
            jax.ShapeDtypeStruct((_N_GENES, 128), jnp.float32),
        ],
    )(emb_t, ct_W, dec_W, dec_b.reshape(1, _N_GENES))


def _take16(win, idx):
    return jnp.take_along_axis(win, idx, axis=0, mode="promise_in_bounds")


def _make_sc_gather():
    info = plsc.get_sparse_core_info()
    nc, ns, nl = info.num_cores, info.num_subcores, info.num_lanes
    nw = nc * ns
    bw = _B // nw
    nlb = _N_LABELS

    mesh = plsc.VectorSubcoreMesh(core_axis_name="c", subcore_axis_name="s")

    scratch = [
        pltpu.VMEM((bw,), jnp.int32),
        pltpu.VMEM((bw,), jnp.int32),
        pltpu.VMEM((bw,), jnp.int32),
        pltpu.VMEM((bw,), jnp.int32),
        pltpu.VMEM((bw * nlb,), jnp.int32),
        pltpu.VMEM((bw, nl), jnp.float32),
        pltpu.VMEM((bw, nl), jnp.float32),
        pltpu.VMEM((bw * nlb,), jnp.float32),
        pltpu.VMEM((bw, nl), jnp.float32),
        pltpu.SemaphoreType.DMA,
        pltpu.SemaphoreType.DMA,
    ]

    @functools.partial(
        pl.kernel,
        mesh=mesh,
        compiler_params=pltpu.CompilerParams(use_tc_tiling_on_sc=False),
        out_type=[
            jax.ShapeDtypeStruct((_B, nl), jnp.float32),
            jax.ShapeDtypeStruct((_B * nlb,), jnp.float32),
            jax.ShapeDtypeStruct((_B, nl), jnp.float32),
        ],
        scratch_types=scratch,
    )
    def sc_kernel(cells_hbm, genes_hbm, pzcw_hbm, p_hbm, pc_hbm,
                  pcz_out, qz_out, pz_out,
                  cells_v, genes_v, krow_v, grow_v, qidx_v, pz_v, pcr_v, qz_v,
                  pcz_v, sem, sem2):
        wid = lax.axis_index("s") * nc + lax.axis_index("c")
        base = wid * bw

        pltpu.sync_copy(cells_hbm.at[pl.ds(base, bw)], cells_v)
        pltpu.sync_copy(genes_hbm.at[pl.ds(base, bw)], genes_v)
        def gr_body(k, _):
            sl = pl.ds(k * nl, nl)
            grow_v[sl] = genes_v[sl] * nlb
            return 0
        lax.fori_loop(0, bw // nl, gr_body, 0, unroll=4)
        cp_pcr = pltpu.async_copy(pc_hbm.at[grow_v], pcr_v, sem2)

        def kr_body(k, _):
            sl = pl.ds(k * nl, nl)
            c = cells_v[sl]
            krow_v[sl] = lax.rem(c, _PS) * 8 + lax.div(c, _PS)
            return 0
        lax.fori_loop(0, bw // nl, kr_body, 0, unroll=4)

        cp_pz = pltpu.async_copy(p_hbm.at[krow_v], pz_v, sem)

        i16 = lax.iota(jnp.int32, nl)
        lsub = i16 & (nlb - 1)
        esub = i16 >> 3

        def qidx_body(k, _):
            win = pl.ds((k >> 3) * nl, nl)
            erl = esub + (k & 7) * 2
            c16 = _take16(cells_v[win], erl)
            g16 = _take16(genes_v[win], erl)
            qidx_v[pl.ds(k * nl, nl)] = c16 * 1024 + lsub * 128 + g16
            return 0
        lax.fori_loop(0, (bw * nlb) // nl, qidx_body, 0, unroll=4)

        cp_qz = pltpu.async_copy(pzcw_hbm.at[qidx_v], qz_v, sem)

        cp_pz.wait()
        cp_pcr.wait()

        def mul_body(k, _):
            pcz_v[k, :] = pz_v[k, :] * pcr_v[k, :]
            return 0
        lax.fori_loop(0, bw, mul_body, 0, unroll=8)

        pltpu.sync_copy(pz_v, pz_out.at[pl.ds(base, bw)])
        pltpu.sync_copy(pcz_v, pcz_out.at[pl.ds(base, bw)])
        cp_qz.wait()
        pltpu.sync_copy(qz_v, qz_out.at[pl.ds(base * nlb, bw * nlb)])

    return sc_kernel


def kernel(cells, genes, emb_table, ct_W, dec_W, dec_b, pz_cw):
    cells = cells.astype(jnp.int32)
    genes = genes.astype(jnp.int32)
    emb_t = emb_table.T
    pzcw_flat = jnp.transpose(pz_cw, (0, 2, 1)).reshape(-1)
    p2, pc128 = _tc_tables(emb_t, ct_W, dec_W, dec_b)
    p_table = p2.reshape(_PS * 8, 16)
    pc_table = pc128.reshape(_N_GENES * 8, 16)
    sc = _make_sc_gather()
    pcz16, qz_f, pz16 = sc(cells, genes, pzcw_flat, p_table, pc_table)
    shp = (_B, _N_LABELS)
    return (pcz16[:, :_N_LABELS], qz_f.reshape(shp), pz16[:, :_N_LABELS])
